# baseline (speedup 1.0000x reference)
import functools

import jax
import jax.numpy as jnp
from jax import lax
from jax.experimental import pallas as pl
from jax.experimental.pallas import tpu as pltpu
from jax.experimental.pallas import tpu_sc as plsc

NU = 50000
NI = 50000
NPAD = 50176
ROWS_PER_TILE = NPAD // 16
SIDE = NPAD
CHUNK = 2 * NPAD
ND = 4
FLAT_ROWS = ND * CHUNK
E = 1_000_000
EPC = 1 << 20
PADE = EPC - E
IDX_ROWS = EPC // 128
TILE_IDX_ROWS = IDX_ROWS // 16
BLK_ROWS = 8
N_BLK = TILE_IDX_ROWS // BLK_ROWS
ZROWS = 64
NZ = ROWS_PER_TILE // ZROWS


def _layer_build():
    mesh = plsc.VectorSubcoreMesh(core_axis_name="c", subcore_axis_name="s")

    @functools.partial(
        pl.kernel,
        mesh=mesh,
        compiler_params=pltpu.CompilerParams(use_tc_tiling_on_sc=False),
        out_type=jax.ShapeDtypeStruct((FLAT_ROWS, 16), jnp.float32),
        scratch_types=[
            pltpu.VMEM_SHARED((NPAD, 16), jnp.float32),
            pltpu.VMEM_SHARED((NPAD, 16), jnp.float32),
            pltpu.VMEM((BLK_ROWS, 128), jnp.int32),
            pltpu.VMEM((BLK_ROWS, 128), jnp.int32),
            pltpu.VMEM((BLK_ROWS * 128, 16), jnp.float32),
            pltpu.VMEM((ZROWS, 16), jnp.float32),
            pltpu.SemaphoreType.DMA,
            pltpu.SemaphoreType.DMA,
        ],
    )
    def layer(gidx, sidx, cur, out, src, acc, gbuf, sbuf, rbuf, zbuf,
              gsem, ssem):
        c = lax.axis_index("c")
        s = lax.axis_index("s")

        def zero_row(i, carry):
            zbuf[i, pl.ds(0, 16)] = jnp.zeros((16,), jnp.float32)
            return carry

        lax.fori_loop(0, ZROWS, zero_row, 0)

        def chunk_body(d, carry):
            src_base = d * CHUNK + (1 - c) * SIDE + s * ROWS_PER_TILE
            pltpu.sync_copy(
                cur.at[pl.ds(src_base, ROWS_PER_TILE)],
                src.at[pl.ds(s * ROWS_PER_TILE, ROWS_PER_TILE)])

            def zcopy(z, carry2):
                pltpu.sync_copy(
                    zbuf, acc.at[pl.ds(s * ROWS_PER_TILE + z * ZROWS, ZROWS)])
                return carry2

            lax.fori_loop(0, NZ, zcopy, 0)
            plsc.subcore_barrier()

            def block(b, carry2):
                off = s * TILE_IDX_ROWS + b * BLK_ROWS
                pltpu.sync_copy(gidx.at[c, pl.ds(off, BLK_ROWS)], gbuf)
                pltpu.sync_copy(sidx.at[c, pl.ds(off, BLK_ROWS)], sbuf)
                gh = []
                for j in range(BLK_ROWS):
                    gh.append(pltpu.async_copy(
                        src.at[gbuf.at[j]],
                        rbuf.at[pl.ds(j * 128, 128)], gsem))
                sh = []
                for j in range(BLK_ROWS):
                    gh[j].wait()
                    sh.append(pltpu.async_copy(
                        rbuf.at[pl.ds(j * 128, 128)],
                        acc.at[sbuf.at[j]], ssem, add=True))
                for h in sh:
                    h.wait()
                return carry2

            lax.fori_loop(0, N_BLK, block, 0)
            plsc.subcore_barrier()

            out_base = d * CHUNK + c * SIDE + s * ROWS_PER_TILE
            pltpu.sync_copy(
                acc.at[pl.ds(s * ROWS_PER_TILE, ROWS_PER_TILE)],
                out.at[pl.ds(out_base, ROWS_PER_TILE)])
            plsc.subcore_barrier()
            return carry

        lax.fori_loop(0, ND, chunk_body, 0)

    return layer


_layer = _layer_build()


def _combine_body(x_ref, a_ref, b_ref, c_ref, o_ref):
    o_ref[...] = (x_ref[...] + a_ref[...] + b_ref[...] + c_ref[...]) * 0.25


def _combine(x0, c1, c2, c3):
    rs = lambda a: a.reshape(6272, 1024)
    spec = pl.BlockSpec((392, 1024), lambda i: (i, 0))
    out = pl.pallas_call(
        _combine_body,
        out_shape=jax.ShapeDtypeStruct((6272, 1024), jnp.float32),
        grid=(16,),
        in_specs=[spec] * 4,
        out_specs=spec,
    )(rs(x0), rs(c1), rs(c2), rs(c3))
    return out.reshape(FLAT_ROWS, 16)


def _chunkify(tab):
    t = tab.reshape(NU, ND, 16).transpose(1, 0, 2)
    return jnp.pad(t, ((0, 0), (0, NPAD - NU), (0, 0)))


def kernel(edge_index, user_table, item_table):
    u = edge_index[:, 0].astype(jnp.int32)
    i = edge_index[:, 1].astype(jnp.int32)

    pad_g = jnp.arange(PADE, dtype=jnp.int32) % 128
    pad_s = NU + (jnp.arange(PADE, dtype=jnp.int32) % 128)
    u_g = jnp.concatenate([u, pad_g])
    i_g = jnp.concatenate([i, pad_g])
    u_s = jnp.concatenate([u, pad_s])
    i_s = jnp.concatenate([i, pad_s])

    gidx = jnp.stack([i_g, u_g]).reshape(2, IDX_ROWS, 128)
    sidx = jnp.stack([u_s, i_s]).reshape(2, IDX_ROWS, 128)

    uc = _chunkify(user_table)
    ic = _chunkify(item_table)
    x0 = jnp.concatenate([uc, ic], axis=1).reshape(FLAT_ROWS, 16)

    c1 = _layer(gidx, sidx, x0)
    c2 = _layer(gidx, sidx, c1)
    c3 = _layer(gidx, sidx, c2)
    fin = _combine(x0, c1, c2, c3)

    f = fin.reshape(ND, 2, NPAD, 16)
    user_f = f[:, 0, :NU, :].transpose(1, 0, 2).reshape(NU, 64)
    item_f = f[:, 1, :NI, :].transpose(1, 0, 2).reshape(NI, 64)
    return (user_f, item_f)


# speedup vs baseline: 1.0343x; 1.0343x over previous
import functools

import jax
import jax.numpy as jnp
from jax import lax
from jax.experimental import pallas as pl
from jax.experimental.pallas import tpu as pltpu
from jax.experimental.pallas import tpu_sc as plsc

NU = 50000
NI = 50000
NPAD = 50176
ROWS_PER_TILE = NPAD // 16
SIDE = NPAD
CHUNK = 2 * NPAD
ND = 4
FLAT_ROWS = ND * CHUNK
E = 1_000_000
EPC = 1 << 20
PADE = EPC - E
IDX_ROWS = EPC // 128
TILE_IDX_ROWS = IDX_ROWS // 16
BLK_ROWS = 8
N_BLK = TILE_IDX_ROWS // BLK_ROWS
N_IT = N_BLK // 2
ZROWS = 64
NZ = ROWS_PER_TILE // ZROWS


def _layer_build():
    mesh = plsc.VectorSubcoreMesh(core_axis_name="c", subcore_axis_name="s")

    @functools.partial(
        pl.kernel,
        mesh=mesh,
        compiler_params=pltpu.CompilerParams(use_tc_tiling_on_sc=False),
        out_type=jax.ShapeDtypeStruct((FLAT_ROWS, 16), jnp.float32),
        scratch_types=[
            pltpu.VMEM_SHARED((NPAD, 16), jnp.float32),
            [pltpu.VMEM((BLK_ROWS, 128), jnp.int32)] * 2,
            [pltpu.VMEM((BLK_ROWS, 128), jnp.int32)] * 2,
            [pltpu.VMEM((BLK_ROWS * 128, 16), jnp.float32)] * 2,
            pltpu.VMEM((ZROWS, 16), jnp.float32),
            pltpu.SemaphoreType.DMA,
            pltpu.SemaphoreType.DMA,
        ],
    )
    def layer(gidx, sidx, cur, out, acc, gbufs, sbufs, rbufs, zbuf,
              gsem, ssem):
        c = lax.axis_index("c")
        s = lax.axis_index("s")

        def zero_row(i, carry):
            zbuf[i, pl.ds(0, 16)] = jnp.zeros((16,), jnp.float32)
            return carry

        lax.fori_loop(0, ZROWS, zero_row, 0)

        def load_idx(k, r, b):
            off = s * TILE_IDX_ROWS + b * BLK_ROWS
            pltpu.sync_copy(gidx.at[r, pl.ds(off, BLK_ROWS)], gbufs[k])
            pltpu.sync_copy(sidx.at[c, pl.ds(off, BLK_ROWS)], sbufs[k])

        def fire_gathers(k):
            return [
                pltpu.async_copy(
                    cur.at[gbufs[k].at[j]],
                    rbufs[k].at[pl.ds(j * 128, 128)], gsem)
                for j in range(BLK_ROWS)
            ]

        def scatter_per_landing(k, gh):
            sh = []
            for j in range(BLK_ROWS):
                gh[j].wait()
                sh.append(pltpu.async_copy(
                    rbufs[k].at[pl.ds(j * 128, 128)],
                    acc.at[sbufs[k].at[j]], ssem, add=True))
            return sh

        def chunk_body(d, carry):
            r = c * ND + d

            def zcopy(z, carry2):
                pltpu.sync_copy(
                    zbuf, acc.at[pl.ds(s * ROWS_PER_TILE + z * ZROWS, ZROWS)])
                return carry2

            lax.fori_loop(0, NZ, zcopy, 0)
            plsc.subcore_barrier()

            def pair(g, carry2):
                load_idx(0, r, 2 * g)
                gh0 = fire_gathers(0)
                load_idx(1, r, 2 * g + 1)
                sh0 = scatter_per_landing(0, gh0)
                gh1 = fire_gathers(1)
                for h in sh0:
                    h.wait()
                sh1 = scatter_per_landing(1, gh1)
                for h in sh1:
                    h.wait()
                return carry2

            lax.fori_loop(0, N_IT, pair, 0)
            plsc.subcore_barrier()

            out_base = d * CHUNK + c * SIDE + s * ROWS_PER_TILE
            rb = BLK_ROWS * 128

            def wb(p, carry2):
                pltpu.sync_copy(
                    acc.at[pl.ds(s * ROWS_PER_TILE + p * rb, rb)], rbufs[0])
                pltpu.sync_copy(
                    rbufs[0], out.at[pl.ds(out_base + p * rb, rb)])
                return carry2

            lax.fori_loop(0, ROWS_PER_TILE // rb, wb, 0)
            rem = ROWS_PER_TILE % rb
            rbase = ROWS_PER_TILE - rem
            pltpu.sync_copy(
                acc.at[pl.ds(s * ROWS_PER_TILE + rbase, rem)],
                rbufs[0].at[pl.ds(0, rem)])
            pltpu.sync_copy(
                rbufs[0].at[pl.ds(0, rem)],
                out.at[pl.ds(out_base + rbase, rem)])
            plsc.subcore_barrier()
            return carry

        lax.fori_loop(0, ND, chunk_body, 0)

    return layer


_layer = _layer_build()


def _combine_body(x_ref, a_ref, b_ref, c_ref, o_ref):
    o_ref[...] = (x_ref[...] + a_ref[...] + b_ref[...] + c_ref[...]) * 0.25


def _combine(x0, c1, c2, c3):
    rs = lambda a: a.reshape(6272, 1024)
    spec = pl.BlockSpec((392, 1024), lambda i: (i, 0))
    out = pl.pallas_call(
        _combine_body,
        out_shape=jax.ShapeDtypeStruct((6272, 1024), jnp.float32),
        grid=(16,),
        in_specs=[spec] * 4,
        out_specs=spec,
    )(rs(x0), rs(c1), rs(c2), rs(c3))
    return out.reshape(FLAT_ROWS, 16)


def _chunkify(tab):
    t = tab.reshape(NU, ND, 16).transpose(1, 0, 2)
    return jnp.pad(t, ((0, 0), (0, NPAD - NU), (0, 0)))


def kernel(edge_index, user_table, item_table):
    u = edge_index[:, 0].astype(jnp.int32)
    i = edge_index[:, 1].astype(jnp.int32)

    pad_g = jnp.arange(PADE, dtype=jnp.int32) % 128
    pad_s = NU + (jnp.arange(PADE, dtype=jnp.int32) % 128)
    u_g = jnp.concatenate([u, pad_g])
    i_g = jnp.concatenate([i, pad_g])
    u_s = jnp.concatenate([u, pad_s])
    i_s = jnp.concatenate([i, pad_s])

    gidx = jnp.stack(
        [d * CHUNK + SIDE + i_g for d in range(ND)]
        + [d * CHUNK + u_g for d in range(ND)]
    ).reshape(2 * ND, IDX_ROWS, 128)
    sidx = jnp.stack([u_s, i_s]).reshape(2, IDX_ROWS, 128)

    uc = _chunkify(user_table)
    ic = _chunkify(item_table)
    x0 = jnp.concatenate([uc, ic], axis=1).reshape(FLAT_ROWS, 16)

    c1 = _layer(gidx, sidx, x0)
    c2 = _layer(gidx, sidx, c1)
    c3 = _layer(gidx, sidx, c2)
    fin = _combine(x0, c1, c2, c3)

    f = fin.reshape(ND, 2, NPAD, 16)
    user_f = f[:, 0, :NU, :].transpose(1, 0, 2).reshape(NU, 64)
    item_f = f[:, 1, :NI, :].transpose(1, 0, 2).reshape(NI, 64)
    return (user_f, item_f)


# speedup vs baseline: 1.1009x; 1.0644x over previous
import functools

import jax
import jax.numpy as jnp
from jax import lax
from jax.experimental import pallas as pl
from jax.experimental.pallas import tpu as pltpu
from jax.experimental.pallas import tpu_sc as plsc

NU = 50000
NI = 50000
NPAD = 50176
ROWS_PER_TILE = NPAD // 16
SIDE = NPAD
CHUNK = 2 * NPAD
ND = 4
FLAT_ROWS = ND * CHUNK
E = 1_000_000
EPC = 1 << 20
PADE = EPC - E
IDX_ROWS = EPC // 128
TILE_IDX_ROWS = IDX_ROWS // 16
BLK_ROWS = 16
N_BLK = TILE_IDX_ROWS // BLK_ROWS
ZROWS = 64
NZ = ROWS_PER_TILE // ZROWS


def _layer_build():
    mesh = plsc.VectorSubcoreMesh(core_axis_name="c", subcore_axis_name="s")

    @functools.partial(
        pl.kernel,
        mesh=mesh,
        compiler_params=pltpu.CompilerParams(use_tc_tiling_on_sc=False),
        out_type=jax.ShapeDtypeStruct((FLAT_ROWS, 16), jnp.float32),
        scratch_types=[
            pltpu.VMEM_SHARED((NPAD, 16), jnp.float32),
            pltpu.VMEM((BLK_ROWS, 128), jnp.int32),
            pltpu.VMEM((BLK_ROWS, 128), jnp.int32),
            pltpu.VMEM((BLK_ROWS * 128, 16), jnp.float32),
            pltpu.VMEM((ZROWS, 16), jnp.float32),
            pltpu.SemaphoreType.DMA,
            pltpu.SemaphoreType.DMA,
        ],
    )
    def layer(gidx, sidx, cur, out, acc, gbuf, sbuf, rbuf, zbuf, gsem, ssem):
        c = lax.axis_index("c")
        s = lax.axis_index("s")

        def zero_row(i, carry):
            zbuf[i, pl.ds(0, 16)] = jnp.zeros((16,), jnp.float32)
            return carry

        lax.fori_loop(0, ZROWS, zero_row, 0)

        def chunk_body(d, carry):
            r = c * ND + d

            def zcopy(z, carry2):
                pltpu.sync_copy(
                    zbuf, acc.at[pl.ds(s * ROWS_PER_TILE + z * ZROWS, ZROWS)])
                return carry2

            lax.fori_loop(0, NZ, zcopy, 0)
            plsc.subcore_barrier()

            def block(b, carry2):
                off = s * TILE_IDX_ROWS + b * BLK_ROWS
                pltpu.sync_copy(gidx.at[r, pl.ds(off, BLK_ROWS)], gbuf)
                pltpu.sync_copy(sidx.at[c, pl.ds(off, BLK_ROWS)], sbuf)
                gh = []
                for j in range(BLK_ROWS):
                    gh.append(pltpu.async_copy(
                        cur.at[gbuf.at[j]],
                        rbuf.at[pl.ds(j * 128, 128)], gsem))
                sh = []
                for j in range(BLK_ROWS):
                    gh[j].wait()
                    sh.append(pltpu.async_copy(
                        rbuf.at[pl.ds(j * 128, 128)],
                        acc.at[sbuf.at[j]], ssem, add=True))
                for h in sh:
                    h.wait()
                return carry2

            lax.fori_loop(0, N_BLK, block, 0)
            plsc.subcore_barrier()

            out_base = d * CHUNK + c * SIDE + s * ROWS_PER_TILE
            rb = 1024

            def wb(p, carry2):
                pltpu.sync_copy(
                    acc.at[pl.ds(s * ROWS_PER_TILE + p * rb, rb)],
                    rbuf.at[pl.ds(0, rb)])
                pltpu.sync_copy(
                    rbuf.at[pl.ds(0, rb)],
                    out.at[pl.ds(out_base + p * rb, rb)])
                return carry2

            lax.fori_loop(0, ROWS_PER_TILE // rb, wb, 0)
            rem = ROWS_PER_TILE % rb
            rbase = ROWS_PER_TILE - rem
            pltpu.sync_copy(
                acc.at[pl.ds(s * ROWS_PER_TILE + rbase, rem)],
                rbuf.at[pl.ds(0, rem)])
            pltpu.sync_copy(
                rbuf.at[pl.ds(0, rem)],
                out.at[pl.ds(out_base + rbase, rem)])
            plsc.subcore_barrier()
            return carry

        lax.fori_loop(0, ND, chunk_body, 0)

    return layer


_layer = _layer_build()


def _combine_body(x_ref, a_ref, b_ref, c_ref, o_ref):
    o_ref[...] = (x_ref[...] + a_ref[...] + b_ref[...] + c_ref[...]) * 0.25


def _combine(x0, c1, c2, c3):
    rs = lambda a: a.reshape(6272, 1024)
    spec = pl.BlockSpec((392, 1024), lambda i: (i, 0))
    out = pl.pallas_call(
        _combine_body,
        out_shape=jax.ShapeDtypeStruct((6272, 1024), jnp.float32),
        grid=(16,),
        in_specs=[spec] * 4,
        out_specs=spec,
    )(rs(x0), rs(c1), rs(c2), rs(c3))
    return out.reshape(FLAT_ROWS, 16)


def _chunkify(tab):
    t = tab.reshape(NU, ND, 16).transpose(1, 0, 2)
    return jnp.pad(t, ((0, 0), (0, NPAD - NU), (0, 0)))


def kernel(edge_index, user_table, item_table):
    u = edge_index[:, 0].astype(jnp.int32)
    i = edge_index[:, 1].astype(jnp.int32)

    pad_g = jnp.arange(PADE, dtype=jnp.int32) % 128
    pad_s = NU + (jnp.arange(PADE, dtype=jnp.int32) % 128)
    u_g = jnp.concatenate([u, pad_g])
    i_g = jnp.concatenate([i, pad_g])
    u_s = jnp.concatenate([u, pad_s])
    i_s = jnp.concatenate([i, pad_s])

    gidx = jnp.stack(
        [d * CHUNK + SIDE + i_g for d in range(ND)]
        + [d * CHUNK + u_g for d in range(ND)]
    ).reshape(2 * ND, IDX_ROWS, 128)
    sidx = jnp.stack([u_s, i_s]).reshape(2, IDX_ROWS, 128)

    uc = _chunkify(user_table)
    ic = _chunkify(item_table)
    x0 = jnp.concatenate([uc, ic], axis=1).reshape(FLAT_ROWS, 16)

    c1 = _layer(gidx, sidx, x0)
    c2 = _layer(gidx, sidx, c1)
    c3 = _layer(gidx, sidx, c2)
    fin = _combine(x0, c1, c2, c3)

    f = fin.reshape(ND, 2, NPAD, 16)
    user_f = f[:, 0, :NU, :].transpose(1, 0, 2).reshape(NU, 64)
    item_f = f[:, 1, :NI, :].transpose(1, 0, 2).reshape(NI, 64)
    return (user_f, item_f)


# speedup vs baseline: 1.2594x; 1.1440x over previous
import functools

import jax
import jax.numpy as jnp
from jax import lax
from jax.experimental import pallas as pl
from jax.experimental.pallas import tpu as pltpu
from jax.experimental.pallas import tpu_sc as plsc

NU = 50000
NI = 50000
NPAD = 50176
ROWS_PER_TILE = NPAD // 16
SIDE = NPAD
CHUNK = 2 * NPAD
ND = 4
FLAT_ROWS = ND * CHUNK
E = 1_000_000
EPC = 1 << 20
PADE = EPC - E
IDX_ROWS = EPC // 128
TILE_IDX_ROWS = IDX_ROWS // 16
BLK_ROWS = 32
N_BLK = TILE_IDX_ROWS // BLK_ROWS
ZROWS = 64
NZ = ROWS_PER_TILE // ZROWS


def _layer_build():
    mesh = plsc.VectorSubcoreMesh(core_axis_name="c", subcore_axis_name="s")

    @functools.partial(
        pl.kernel,
        mesh=mesh,
        compiler_params=pltpu.CompilerParams(use_tc_tiling_on_sc=False),
        out_type=jax.ShapeDtypeStruct((FLAT_ROWS, 16), jnp.float32),
        scratch_types=[
            pltpu.VMEM_SHARED((NPAD, 16), jnp.float32),
            pltpu.VMEM((BLK_ROWS, 128), jnp.int32),
            pltpu.VMEM((BLK_ROWS, 128), jnp.int32),
            pltpu.VMEM((BLK_ROWS * 128, 16), jnp.float32),
            pltpu.VMEM((ZROWS, 16), jnp.float32),
            pltpu.SemaphoreType.DMA,
            pltpu.SemaphoreType.DMA,
        ],
    )
    def layer(gidx, sidx, cur, out, acc, gbuf, sbuf, rbuf, zbuf, gsem, ssem):
        c = lax.axis_index("c")
        s = lax.axis_index("s")

        def zero_row(i, carry):
            zbuf[i, pl.ds(0, 16)] = jnp.zeros((16,), jnp.float32)
            return carry

        lax.fori_loop(0, ZROWS, zero_row, 0)

        def chunk_body(d, carry):
            r = c * ND + d

            def zcopy(z, carry2):
                pltpu.sync_copy(
                    zbuf, acc.at[pl.ds(s * ROWS_PER_TILE + z * ZROWS, ZROWS)])
                return carry2

            lax.fori_loop(0, NZ, zcopy, 0)
            plsc.subcore_barrier()

            def block(b, carry2):
                off = s * TILE_IDX_ROWS + b * BLK_ROWS
                pltpu.sync_copy(gidx.at[r, pl.ds(off, BLK_ROWS)], gbuf)
                pltpu.sync_copy(sidx.at[c, pl.ds(off, BLK_ROWS)], sbuf)
                gh = []
                for j in range(BLK_ROWS):
                    gh.append(pltpu.async_copy(
                        cur.at[gbuf.at[j]],
                        rbuf.at[pl.ds(j * 128, 128)], gsem))
                sh = []
                for j in range(BLK_ROWS):
                    gh[j].wait()
                    sh.append(pltpu.async_copy(
                        rbuf.at[pl.ds(j * 128, 128)],
                        acc.at[sbuf.at[j]], ssem, add=True))
                for h in sh:
                    h.wait()
                return carry2

            lax.fori_loop(0, N_BLK, block, 0)
            plsc.subcore_barrier()

            out_base = d * CHUNK + c * SIDE + s * ROWS_PER_TILE
            rb = 1024

            def wb(p, carry2):
                pltpu.sync_copy(
                    acc.at[pl.ds(s * ROWS_PER_TILE + p * rb, rb)],
                    rbuf.at[pl.ds(0, rb)])
                pltpu.sync_copy(
                    rbuf.at[pl.ds(0, rb)],
                    out.at[pl.ds(out_base + p * rb, rb)])
                return carry2

            lax.fori_loop(0, ROWS_PER_TILE // rb, wb, 0)
            rem = ROWS_PER_TILE % rb
            rbase = ROWS_PER_TILE - rem
            pltpu.sync_copy(
                acc.at[pl.ds(s * ROWS_PER_TILE + rbase, rem)],
                rbuf.at[pl.ds(0, rem)])
            pltpu.sync_copy(
                rbuf.at[pl.ds(0, rem)],
                out.at[pl.ds(out_base + rbase, rem)])
            plsc.subcore_barrier()
            return carry

        lax.fori_loop(0, ND, chunk_body, 0)

    return layer


_layer = _layer_build()


def _combine_body(x_ref, a_ref, b_ref, c_ref, o_ref):
    o_ref[...] = (x_ref[...] + a_ref[...] + b_ref[...] + c_ref[...]) * 0.25


def _combine(x0, c1, c2, c3):
    rs = lambda a: a.reshape(6272, 1024)
    spec = pl.BlockSpec((392, 1024), lambda i: (i, 0))
    out = pl.pallas_call(
        _combine_body,
        out_shape=jax.ShapeDtypeStruct((6272, 1024), jnp.float32),
        grid=(16,),
        in_specs=[spec] * 4,
        out_specs=spec,
    )(rs(x0), rs(c1), rs(c2), rs(c3))
    return out.reshape(FLAT_ROWS, 16)


def _chunkify(tab):
    t = tab.reshape(NU, ND, 16).transpose(1, 0, 2)
    return jnp.pad(t, ((0, 0), (0, NPAD - NU), (0, 0)))


def kernel(edge_index, user_table, item_table):
    u = edge_index[:, 0].astype(jnp.int32)
    i = edge_index[:, 1].astype(jnp.int32)

    pad_g = jnp.arange(PADE, dtype=jnp.int32) % 128
    pad_s = NU + (jnp.arange(PADE, dtype=jnp.int32) % 128)
    u_g = jnp.concatenate([u, pad_g])
    i_g = jnp.concatenate([i, pad_g])
    u_s = jnp.concatenate([u, pad_s])
    i_s = jnp.concatenate([i, pad_s])

    gidx = jnp.stack(
        [d * CHUNK + SIDE + i_g for d in range(ND)]
        + [d * CHUNK + u_g for d in range(ND)]
    ).reshape(2 * ND, IDX_ROWS, 128)
    sidx = jnp.stack([u_s, i_s]).reshape(2, IDX_ROWS, 128)

    uc = _chunkify(user_table)
    ic = _chunkify(item_table)
    x0 = jnp.concatenate([uc, ic], axis=1).reshape(FLAT_ROWS, 16)

    c1 = _layer(gidx, sidx, x0)
    c2 = _layer(gidx, sidx, c1)
    c3 = _layer(gidx, sidx, c2)
    fin = _combine(x0, c1, c2, c3)

    f = fin.reshape(ND, 2, NPAD, 16)
    user_f = f[:, 0, :NU, :].transpose(1, 0, 2).reshape(NU, 64)
    item_f = f[:, 1, :NI, :].transpose(1, 0, 2).reshape(NI, 64)
    return (user_f, item_f)
